# baseline (speedup 1.0000x reference)
import functools

import jax
import jax.numpy as jnp
from jax import lax
from jax.experimental import pallas as pl
from jax.experimental.pallas import tpu as pltpu
from jax.experimental.pallas import tpu_sc as plsc

LANES = 16
PBLK = 128


def _emb_kernel(B, T, D, idx_hbm, tok_hbm, pos_hbm, out_hbm,
                idx_v, pos_v, rows_v, sem):
    NPB = T // PBLK
    BH = B // 2
    wid = lax.axis_index("s") * 2 + lax.axis_index("c")
    pb = wid % NPB
    h = wid // NPB
    p0 = pb * PBLK
    b0 = h * BH

    pltpu.sync_copy(pos_hbm.at[pl.ds(p0, PBLK)], pos_v)
    pltpu.sync_copy(idx_hbm.at[pl.ds(b0, BH), pl.ds(p0, PBLK)], idx_v)

    @pl.loop(0, BH)
    def _batch(b):
        pltpu.async_copy(tok_hbm.at[idx_v.at[b]], rows_v, sem).wait()

        @pl.loop(0, PBLK)
        def _row(r):
            for c in range(D // LANES):
                sl = pl.ds(c * LANES, LANES)
                rows_v[r, sl] = rows_v[r, sl] + pos_v[r, sl]

        pltpu.sync_copy(rows_v, out_hbm.at[b0 + b, pl.ds(p0, PBLK)])


def kernel(idx, tok_table, pos_table):
    B, T = idx.shape
    V, D = tok_table.shape
    idx = idx.astype(jnp.int32)

    mesh = plsc.VectorSubcoreMesh(core_axis_name="c", subcore_axis_name="s")

    k = pl.kernel(
        functools.partial(_emb_kernel, B, T, D),
        out_type=jax.ShapeDtypeStruct((B, T, D), jnp.float32),
        mesh=mesh,
        scratch_types=[
            pltpu.VMEM((B // 2, PBLK), jnp.int32),
            pltpu.VMEM((PBLK, D), jnp.float32),
            pltpu.VMEM((PBLK, D), jnp.float32),
            pltpu.SemaphoreType.DMA,
        ],
        compiler_params=pltpu.CompilerParams(use_tc_tiling_on_sc=False),
    )
    return k(idx, tok_table, pos_table)
